# Initial kernel scaffold; baseline (speedup 1.0000x reference)
import jax
import jax.numpy as jnp
from jax.experimental import pallas as pl


def kernel(x, edge_index, W_mu, b_mu, W_logstd, b_logstd):
    raise NotImplementedError("write your pallas kernel here")


# speedup vs baseline: 34.2015x; 34.2015x over previous
import functools

import jax
import jax.numpy as jnp
from jax import lax
from jax.experimental import pallas as pl
from jax.experimental.pallas import tpu as pltpu
from jax.experimental.pallas import tpu_sc as plsc

N_NODES = 10000
D_IN = 128
D_OUT = 32
D_CAT = 2 * D_OUT
N_EDGES = 320000

NC = 2
NS = 16
NW = NC * NS

BB = 128
NB = -(-N_EDGES // (NW * BB))
EPT = NB * BB
E_PAD = NW * EPT

N_PAD = 10240
ROWS_PER_TILE = N_PAD // NS
ZCHUNK = 128

_mesh = plsc.VectorSubcoreMesh(
    core_axis_name="c", subcore_axis_name="s", num_cores=NC, num_subcores=NS
)


def _deg_body(dst_hbm, z_hbm, ones_hbm, degp_hbm, idx_v, zb_v, ones_v, deg_sp):
    c = lax.axis_index("c")
    s = lax.axis_index("s")
    w = c * NS + s
    pltpu.sync_copy(dst_hbm.at[w], idx_v)
    pltpu.sync_copy(z_hbm, zb_v)
    pltpu.sync_copy(ones_hbm, ones_v)
    pltpu.sync_copy(zb_v, deg_sp.at[pl.ds(s * ROWS_PER_TILE, ROWS_PER_TILE)])
    plsc.subcore_barrier()

    def it(j, carry):
        pltpu.sync_copy(ones_v, deg_sp.at[idx_v.at[j]], add=True)
        return carry

    lax.fori_loop(0, NB, it, 0)
    plsc.subcore_barrier()
    pltpu.sync_copy(deg_sp.at[pl.ds(s * ROWS_PER_TILE, ROWS_PER_TILE)], zb_v)
    pltpu.sync_copy(zb_v, degp_hbm.at[c, pl.ds(s * ROWS_PER_TILE, ROWS_PER_TILE)])


_deg_kernel = pl.kernel(
    _deg_body,
    out_type=jax.ShapeDtypeStruct((NC, N_PAD), jnp.float32),
    mesh=_mesh,
    scratch_types=[
        pltpu.VMEM((NB, BB), jnp.int32),
        pltpu.VMEM((ROWS_PER_TILE,), jnp.float32),
        pltpu.VMEM((BB,), jnp.float32),
        pltpu.VMEM_SHARED((N_PAD,), jnp.float32),
    ],
)


def _pre_body(x_ref, w_ref, degp_ref, g_ref):
    deg = degp_ref[0] + degp_ref[1] + 1.0
    dinv = lax.rsqrt(deg)
    h = jnp.dot(x_ref[...], w_ref[...], preferred_element_type=jnp.float32)
    g_ref[...] = h * dinv[:, None]


def _pre(x_pad, w_cat, degp):
    blk = 1024
    return pl.pallas_call(
        _pre_body,
        grid=(N_PAD // blk,),
        in_specs=[
            pl.BlockSpec((blk, D_IN), lambda i: (i, 0)),
            pl.BlockSpec((D_IN, D_CAT), lambda i: (0, 0)),
            pl.BlockSpec((NC, blk), lambda i: (0, i)),
        ],
        out_specs=pl.BlockSpec((blk, D_CAT), lambda i: (i, 0)),
        out_shape=jax.ShapeDtypeStruct((N_PAD, D_CAT), jnp.float32),
    )(x_pad, w_cat, degp)


def _segsum_body(src_hbm, dst_hbm, g_hbm, z_hbm, accp_hbm,
                 sidx_v, didx_v, rows_v, acc_sp, sem):
    c = lax.axis_index("c")
    s = lax.axis_index("s")
    w = c * NS + s
    pltpu.sync_copy(src_hbm.at[w], sidx_v)
    pltpu.sync_copy(dst_hbm.at[w], didx_v)
    pltpu.sync_copy(z_hbm, rows_v)
    base = s * ROWS_PER_TILE
    for r in range(ROWS_PER_TILE // ZCHUNK):
        pltpu.sync_copy(rows_v, acc_sp.at[pl.ds(base + r * ZCHUNK, ZCHUNK)])
    plsc.subcore_barrier()

    def it(j, carry):
        pltpu.async_copy(g_hbm.at[sidx_v.at[j]], rows_v, sem).wait()
        pltpu.sync_copy(rows_v, acc_sp.at[didx_v.at[j]], add=True)
        return carry

    lax.fori_loop(0, NB, it, 0)
    plsc.subcore_barrier()
    for r in range(ROWS_PER_TILE // ZCHUNK):
        pltpu.sync_copy(acc_sp.at[pl.ds(base + r * ZCHUNK, ZCHUNK)], rows_v)
        pltpu.sync_copy(rows_v, accp_hbm.at[c, pl.ds(base + r * ZCHUNK, ZCHUNK)])


_segsum_kernel = pl.kernel(
    _segsum_body,
    out_type=jax.ShapeDtypeStruct((NC, N_PAD, D_CAT), jnp.float32),
    mesh=_mesh,
    scratch_types=[
        pltpu.VMEM((NB, BB), jnp.int32),
        pltpu.VMEM((NB, BB), jnp.int32),
        pltpu.VMEM((BB, D_CAT), jnp.float32),
        pltpu.VMEM_SHARED((N_PAD, D_CAT), jnp.float32),
        pltpu.SemaphoreType.DMA,
    ],
    compiler_params=pltpu.CompilerParams(use_tc_tiling_on_sc=False),
)


def _post_body(accp_ref, g_ref, degp_ref, bmu_ref, bls_ref, mu_ref, ls_ref):
    deg = degp_ref[0] + degp_ref[1] + 1.0
    dinv = lax.rsqrt(deg)
    ssum = accp_ref[0] + accp_ref[1] + g_ref[...]
    o = ssum * dinv[:, None]
    mu_ref[...] = o[:, :D_OUT] + bmu_ref[...]
    ls_ref[...] = o[:, D_OUT:] + bls_ref[...]


def _post(accp, g, degp, b_mu2, b_ls2):
    blk = 1024
    return pl.pallas_call(
        _post_body,
        grid=(N_PAD // blk,),
        in_specs=[
            pl.BlockSpec((NC, blk, D_CAT), lambda i: (0, i, 0)),
            pl.BlockSpec((blk, D_CAT), lambda i: (i, 0)),
            pl.BlockSpec((NC, blk), lambda i: (0, i)),
            pl.BlockSpec((1, D_OUT), lambda i: (0, 0)),
            pl.BlockSpec((1, D_OUT), lambda i: (0, 0)),
        ],
        out_specs=[
            pl.BlockSpec((blk, D_OUT), lambda i: (i, 0)),
            pl.BlockSpec((blk, D_OUT), lambda i: (i, 0)),
        ],
        out_shape=[
            jax.ShapeDtypeStruct((N_PAD, D_OUT), jnp.float32),
            jax.ShapeDtypeStruct((N_PAD, D_OUT), jnp.float32),
        ],
    )(accp, g, degp, b_mu2, b_ls2)


@jax.jit
def kernel(x, edge_index, W_mu, b_mu, W_logstd, b_logstd):
    src = edge_index[0]
    dst = edge_index[1]
    pad = E_PAD - N_EDGES
    src_p = jnp.concatenate([src, jnp.zeros((pad,), jnp.int32)]).reshape(NW, NB, BB)
    dst_p = jnp.concatenate([dst, jnp.full((pad,), N_NODES, jnp.int32)]).reshape(NW, NB, BB)

    w_cat = jnp.concatenate([W_mu, W_logstd], axis=1)
    x_pad = jnp.pad(x, ((0, N_PAD - N_NODES), (0, 0)))
    zeros1 = jnp.zeros((ROWS_PER_TILE,), jnp.float32)
    ones1 = jnp.ones((BB,), jnp.float32)
    zeros2 = jnp.zeros((ZCHUNK, D_CAT), jnp.float32)

    degp = _deg_kernel(dst_p, zeros1, ones1)
    g = _pre(x_pad, w_cat, degp)
    accp = _segsum_kernel(src_p, dst_p, g, zeros2)
    mu_p, ls_p = _post(accp, g, degp, b_mu.reshape(1, D_OUT), b_logstd.reshape(1, D_OUT))
    return mu_p[:N_NODES], ls_p[:N_NODES]
